# baseline (speedup 1.0000x reference)
import functools

import jax
import jax.numpy as jnp
from jax import lax
from jax.experimental import pallas as pl
from jax.experimental.pallas import tpu as pltpu
from jax.experimental.pallas import tpu_sc as plsc

N = 10000
NP = 10240
D_IN = 128
D_PAD = 16

NC = 2
NS = 16
NW = NC * NS
BLK = 128


def _mm_body(x_ref, w_ref, o_ref):
    o_ref[pl.ds(0, N), :] = jnp.dot(x_ref[...], w_ref[...],
                                    preferred_element_type=jnp.float32)


def _matmul(x, w_pad):
    return pl.pallas_call(
        _mm_body,
        out_shape=jax.ShapeDtypeStruct((NP, D_PAD), jnp.float32),
    )(x, w_pad)


def _red_body(p_hbm, b_hbm, o_hbm, v0, v1, bv):
    c = lax.axis_index("c")
    s = lax.axis_index("s")
    wid = c * NS + s
    rows2 = NP // NW
    rb = wid * rows2
    pltpu.sync_copy(p_hbm.at[0, pl.ds(rb, rows2)], v0)
    pltpu.sync_copy(p_hbm.at[1, pl.ds(rb, rows2)], v1)
    pltpu.sync_copy(b_hbm, bv)
    bvec = bv[0, :]

    def _r(i, _):
        v0[i, :] = v0[i, :] + v1[i, :] + bvec
        return 0
    lax.fori_loop(0, rows2, _r, 0)
    pltpu.sync_copy(v0, o_hbm.at[pl.ds(rb, rows2)])


def _reduce_partials(partials, b):
    d_out = b.shape[0]
    bp = jnp.zeros((1, D_PAD), jnp.float32).at[0, :d_out].set(b)
    mesh = plsc.VectorSubcoreMesh(core_axis_name="c", subcore_axis_name="s")
    rows2 = NP // NW
    full = pl.kernel(
        _red_body,
        out_type=jax.ShapeDtypeStruct((NP, D_PAD), jnp.float32),
        mesh=mesh,
        compiler_params=pltpu.CompilerParams(use_tc_tiling_on_sc=False),
        scratch_types=[
            pltpu.VMEM((rows2, D_PAD), jnp.float32),
            pltpu.VMEM((rows2, D_PAD), jnp.float32),
            pltpu.VMEM((1, D_PAD), jnp.float32),
        ],
    )(partials, bp)
    return full[:N, :d_out]


NBUF = 8
LEAD = 4


def _sc_body(h_hbm, src_hbm, dst_hbm, w_hbm, out_hbm,
             bufs, src_v, dst_v, w_v, acc_sh, gsems, ssems, base, extra):
    c = lax.axis_index("c")
    s = lax.axis_index("s")
    wid = c * NS + s

    zeros16 = jnp.zeros((16,), jnp.float32)

    start_blk = wid * base + jnp.minimum(wid, extra)
    pltpu.sync_copy(src_hbm.at[pl.ds(start_blk, base)],
                    src_v.at[pl.ds(0, base)])
    pltpu.sync_copy(dst_hbm.at[pl.ds(start_blk, base)],
                    dst_v.at[pl.ds(0, base)])
    pltpu.sync_copy(w_hbm.at[pl.ds(start_blk, base)],
                    w_v.at[pl.ds(0, base)])
    if extra:
        @pl.when(wid < extra)
        def _():
            eb = start_blk + base
            pltpu.sync_copy(src_hbm.at[pl.ds(eb, 1)],
                            src_v.at[pl.ds(base, 1)])
            pltpu.sync_copy(dst_hbm.at[pl.ds(eb, 1)],
                            dst_v.at[pl.ds(base, 1)])
            pltpu.sync_copy(w_hbm.at[pl.ds(eb, 1)],
                            w_v.at[pl.ds(base, 1)])

    b0 = bufs[0]

    def _z(i, _):
        b0[i, :] = zeros16
        return 0
    lax.fori_loop(0, BLK, _z, 0)

    rows = NP // NS
    rbase = s * rows

    def _zacc(k, _):
        pltpu.sync_copy(b0, acc_sh.at[pl.ds(rbase + k * BLK, BLK)])
        return 0
    lax.fori_loop(0, rows // BLK, _zacc, 0)
    plsc.subcore_barrier()

    def start_gather(jb, buf, sem):
        pltpu.async_copy(h_hbm.at[src_v.at[jb]], buf, sem)

    def wait_gather(jb, buf, sem):
        pltpu.make_async_copy(h_hbm.at[src_v.at[jb]], buf, sem).wait()

    def start_scatter(jb, buf, sem):
        pltpu.async_copy(buf, acc_sh.at[dst_v.at[jb]], sem, add=True)

    def wait_scatter(jb, buf, sem):
        pltpu.make_async_copy(buf, acc_sh.at[dst_v.at[jb]], sem).wait()

    def multiply(buf, jb):
        for g in range(BLK // 16):
            wvec = w_v[jb, pl.ds(g * 16, 16)]
            for k in range(16):
                wb = jnp.take_along_axis(
                    wvec, jnp.full((16,), k, jnp.int32), axis=0)
                o = g * 16 + k
                buf[o, :] = buf[o, :] * wb

    def process(jb, r):
        s2 = (r + LEAD) % NBUF
        wait_gather(jb, bufs[r], gsems[r])
        multiply(bufs[r], jb)
        start_scatter(jb, bufs[r], ssems[r])

        @pl.when(jb + LEAD < base)
        def _():
            @pl.when(jb - LEAD >= 0)
            def _():
                wait_scatter(jb - LEAD, bufs[s2], ssems[s2])
            start_gather(jb + LEAD, bufs[s2], gsems[s2])

    for r in range(LEAD):
        start_gather(jnp.int32(r), bufs[r], gsems[r])

    def _oct(j8, _):
        for r in range(NBUF):
            process(j8 * NBUF + r, r)
        return 0
    no = base // NBUF
    lax.fori_loop(0, no, _oct, 0)
    for r in range(base % NBUF):
        process(jnp.int32(no * NBUF + r), r)
    for t in range(min(2 * LEAD, base)):
        blk = base - min(2 * LEAD, base) + t
        wait_scatter(jnp.int32(blk), bufs[blk % NBUF], ssems[blk % NBUF])

    if extra:
        @pl.when(wid < extra)
        def _():
            jb = jnp.int32(base)
            start_gather(jb, bufs[0], gsems[0])
            wait_gather(jb, bufs[0], gsems[0])
            multiply(bufs[0], jb)
            start_scatter(jb, bufs[0], ssems[0])
            wait_scatter(jb, bufs[0], ssems[0])

    plsc.subcore_barrier()
    pltpu.sync_copy(acc_sh.at[pl.ds(rbase, rows)],
                    out_hbm.at[c, pl.ds(rbase, rows)])


def _edge_aggregate(h, src, dst, w, base, extra):
    mesh = plsc.VectorSubcoreMesh(core_axis_name="c", subcore_axis_name="s")
    body = functools.partial(_sc_body, base=base, extra=extra)
    nslab = base + (1 if extra else 0)
    return pl.kernel(
        body,
        out_type=jax.ShapeDtypeStruct((NC, NP, D_PAD), jnp.float32),
        mesh=mesh,
        compiler_params=pltpu.CompilerParams(use_tc_tiling_on_sc=False),
        scratch_types=[
            [pltpu.VMEM((BLK, D_PAD), jnp.float32) for _ in range(NBUF)],
            pltpu.VMEM((nslab, BLK), jnp.int32),
            pltpu.VMEM((nslab, BLK), jnp.int32),
            pltpu.VMEM((nslab, BLK), jnp.float32),
            pltpu.VMEM_SHARED((NP, D_PAD), jnp.float32),
            [pltpu.SemaphoreType.DMA for _ in range(NBUF)],
            [pltpu.SemaphoreType.DMA for _ in range(NBUF)],
        ],
    )(h, src, dst, w)


@jax.jit
def kernel(x, edge_index, w_mul, W, b):
    e = edge_index.shape[1]
    src = edge_index[0].astype(jnp.int32)
    dst = edge_index[1].astype(jnp.int32)
    w = w_mul
    if e % BLK:
        pad = BLK - e % BLK
        src = jnp.concatenate([src, jnp.zeros((pad,), jnp.int32)])
        dst = jnp.concatenate([dst, jnp.zeros((pad,), jnp.int32)])
        w = jnp.concatenate([w, jnp.zeros((pad,), jnp.float32)])
        e += pad
    nblk = e // BLK
    base, extra = divmod(nblk, NW)
    src = src.reshape(nblk, BLK)
    dst = dst.reshape(nblk, BLK)
    w = w.reshape(nblk, BLK)

    w_pad = jnp.zeros((D_IN, D_PAD), jnp.float32).at[:, : W.shape[1]].set(W)
    h = _matmul(x, w_pad)

    partials = _edge_aggregate(h, src, dst, w, base, extra)
    return _reduce_partials(partials, b)


# speedup vs baseline: 1.0595x; 1.0595x over previous
import functools

import jax
import jax.numpy as jnp
from jax import lax
from jax.experimental import pallas as pl
from jax.experimental.pallas import tpu as pltpu
from jax.experimental.pallas import tpu_sc as plsc

N = 10000
NP = 10240
D_IN = 128
D_PAD = 16

NC = 2
NS = 16
NW = NC * NS
BLK = 128


def _mm_body(x_ref, w_ref, e_ref, o_ref, src_ref, dst_ref):
    o_ref[pl.ds(0, N), :] = jnp.dot(x_ref[...], w_ref[...],
                                    preferred_element_type=jnp.float32)
    e = e_ref[...]
    nblk = e.shape[1] // BLK
    src_ref[...] = e[0].reshape(nblk, BLK)
    dst_ref[...] = e[1].reshape(nblk, BLK)


def _matmul(x, w_pad, edges):
    nblk = edges.shape[1] // BLK
    return pl.pallas_call(
        _mm_body,
        out_shape=(
            jax.ShapeDtypeStruct((NP, D_PAD), jnp.float32),
            jax.ShapeDtypeStruct((nblk, BLK), jnp.int32),
            jax.ShapeDtypeStruct((nblk, BLK), jnp.int32),
        ),
    )(x, w_pad, edges)


NBUF = 8
LEAD = 4


def _sc_body(h_hbm, src_hbm, dst_hbm, w_hbm, out_hbm,
             bufs, src_v, dst_v, w_v, h_sh, acc_sh, gsems, ssems,
             base, extra):
    c = lax.axis_index("c")
    s = lax.axis_index("s")
    wid = c * NS + s

    zeros16 = jnp.zeros((16,), jnp.float32)

    start_blk = wid * base + jnp.minimum(wid, extra)
    pltpu.sync_copy(src_hbm.at[pl.ds(start_blk, base)],
                    src_v.at[pl.ds(0, base)])
    pltpu.sync_copy(dst_hbm.at[pl.ds(start_blk, base)],
                    dst_v.at[pl.ds(0, base)])
    pltpu.sync_copy(w_hbm.at[pl.ds(start_blk, base)],
                    w_v.at[pl.ds(0, base)])
    if extra:
        @pl.when(wid < extra)
        def _():
            eb = start_blk + base
            pltpu.sync_copy(src_hbm.at[pl.ds(eb, 1)],
                            src_v.at[pl.ds(base, 1)])
            pltpu.sync_copy(dst_hbm.at[pl.ds(eb, 1)],
                            dst_v.at[pl.ds(base, 1)])
            pltpu.sync_copy(w_hbm.at[pl.ds(eb, 1)],
                            w_v.at[pl.ds(base, 1)])

    b0 = bufs[0]

    def _z(i, _):
        b0[i, :] = zeros16
        return 0
    lax.fori_loop(0, BLK, _z, 0)

    rows = NP // NS
    rbase = s * rows

    def _zacc(k, _):
        pltpu.sync_copy(b0, acc_sh.at[pl.ds(rbase + k * BLK, BLK)])
        return 0
    lax.fori_loop(0, rows // BLK, _zacc, 0)
    pltpu.sync_copy(h_hbm.at[pl.ds(rbase, rows)], h_sh.at[pl.ds(rbase, rows)])
    plsc.subcore_barrier()

    def start_gather(jb, buf, sem):
        pltpu.async_copy(h_sh.at[src_v.at[jb]], buf, sem)

    def wait_gather(jb, buf, sem):
        pltpu.make_async_copy(h_sh.at[src_v.at[jb]], buf, sem).wait()

    def start_scatter(jb, buf, sem):
        pltpu.async_copy(buf, acc_sh.at[dst_v.at[jb]], sem, add=True)

    def wait_scatter(jb, buf, sem):
        pltpu.make_async_copy(buf, acc_sh.at[dst_v.at[jb]], sem).wait()

    def multiply(buf, jb):
        for g in range(BLK // 16):
            wvec = w_v[jb, pl.ds(g * 16, 16)]
            for k in range(16):
                wb = jnp.take_along_axis(
                    wvec, jnp.full((16,), k, jnp.int32), axis=0)
                o = g * 16 + k
                buf[o, :] = buf[o, :] * wb

    def process(jb, r):
        s2 = (r + LEAD) % NBUF
        wait_gather(jb, bufs[r], gsems[r])
        multiply(bufs[r], jb)
        start_scatter(jb, bufs[r], ssems[r])

        @pl.when(jb + LEAD < base)
        def _():
            @pl.when(jb - LEAD >= 0)
            def _():
                wait_scatter(jb - LEAD, bufs[s2], ssems[s2])
            start_gather(jb + LEAD, bufs[s2], gsems[s2])

    for r in range(LEAD):
        start_gather(jnp.int32(r), bufs[r], gsems[r])

    def _oct(j8, _):
        for r in range(NBUF):
            process(j8 * NBUF + r, r)
        return 0
    no = base // NBUF
    lax.fori_loop(0, no, _oct, 0)
    for r in range(base % NBUF):
        process(jnp.int32(no * NBUF + r), r)
    for t in range(min(2 * LEAD, base)):
        blk = base - min(2 * LEAD, base) + t
        wait_scatter(jnp.int32(blk), bufs[blk % NBUF], ssems[blk % NBUF])

    if extra:
        @pl.when(wid < extra)
        def _():
            jb = jnp.int32(base)
            start_gather(jb, bufs[0], gsems[0])
            wait_gather(jb, bufs[0], gsems[0])
            multiply(bufs[0], jb)
            start_scatter(jb, bufs[0], ssems[0])
            wait_scatter(jb, bufs[0], ssems[0])

    plsc.subcore_barrier()
    pltpu.sync_copy(acc_sh.at[pl.ds(rbase, rows)],
                    out_hbm.at[c, pl.ds(rbase, rows)])


def _edge_aggregate(h, src, dst, w, base, extra):
    mesh = plsc.VectorSubcoreMesh(core_axis_name="c", subcore_axis_name="s")
    body = functools.partial(_sc_body, base=base, extra=extra)
    nslab = base + (1 if extra else 0)
    return pl.kernel(
        body,
        out_type=jax.ShapeDtypeStruct((NC, NP, D_PAD), jnp.float32),
        mesh=mesh,
        compiler_params=pltpu.CompilerParams(use_tc_tiling_on_sc=False),
        scratch_types=[
            [pltpu.VMEM((BLK, D_PAD), jnp.float32) for _ in range(NBUF)],
            pltpu.VMEM((nslab, BLK), jnp.int32),
            pltpu.VMEM((nslab, BLK), jnp.int32),
            pltpu.VMEM((nslab, BLK), jnp.float32),
            pltpu.VMEM_SHARED((NP, D_PAD), jnp.float32),
            pltpu.VMEM_SHARED((NP, D_PAD), jnp.float32),
            [pltpu.SemaphoreType.DMA for _ in range(NBUF)],
            [pltpu.SemaphoreType.DMA for _ in range(NBUF)],
        ],
    )(h, src, dst, w)


@jax.jit
def kernel(x, edge_index, w_mul, W, b):
    e = edge_index.shape[1]
    edges = edge_index.astype(jnp.int32)
    w = w_mul
    if e % BLK:
        pad = BLK - e % BLK
        edges = jnp.concatenate(
            [edges, jnp.zeros((2, pad), jnp.int32)], axis=1)
        w = jnp.concatenate([w, jnp.zeros((pad,), jnp.float32)])
        e += pad
    nblk = e // BLK
    base, extra = divmod(nblk, NW)
    w = w.reshape(nblk, BLK)

    w_pad = jnp.zeros((D_IN, D_PAD), jnp.float32).at[:, : W.shape[1]].set(W)
    h, src, dst = _matmul(x, w_pad, edges)

    partials = _edge_aggregate(h, src, dst, w, base, extra)
    pf = partials.reshape(NC, NP * D_PAD)
    out = (pf[0] + pf[1]).reshape(NP, D_PAD)
    return out[:N, : W.shape[1]] + b


# speedup vs baseline: 1.1594x; 1.0943x over previous
import functools

import jax
import jax.numpy as jnp
from jax import lax
from jax.experimental import pallas as pl
from jax.experimental.pallas import tpu as pltpu
from jax.experimental.pallas import tpu_sc as plsc

N = 10000
NP = 10240
D_IN = 128
D_PAD = 16

NC = 2
NS = 16
NW = NC * NS
BLK = 128


def _mm_body(x_ref, w_ref, e_ref, o_ref, src_ref, dst_ref):
    o_ref[pl.ds(0, N), :] = jnp.dot(x_ref[...], w_ref[...],
                                    preferred_element_type=jnp.float32)
    e = e_ref[...]
    nblk = e.shape[1] // BLK
    src_ref[...] = e[0].reshape(nblk, BLK)
    dst_ref[...] = e[1].reshape(nblk, BLK)


def _matmul(x, w_pad, edges):
    nblk = edges.shape[1] // BLK
    return pl.pallas_call(
        _mm_body,
        out_shape=(
            jax.ShapeDtypeStruct((NP, D_PAD), jnp.float32),
            jax.ShapeDtypeStruct((nblk, BLK), jnp.int32),
            jax.ShapeDtypeStruct((nblk, BLK), jnp.int32),
        ),
    )(x, w_pad, edges)


NBUF = 4
LEAD = 2


def _sc_body(h_hbm, src_hbm, dst_hbm, w_hbm, out_hbm,
             bufs, src_v, dst_v, w_v, h_sh, acc_sh, gsems, ssems,
             base, extra):
    c = lax.axis_index("c")
    s = lax.axis_index("s")
    wid = c * NS + s

    zeros16 = jnp.zeros((16,), jnp.float32)

    start_blk = wid * base + jnp.minimum(wid, extra)
    pltpu.sync_copy(src_hbm.at[pl.ds(start_blk, base)],
                    src_v.at[pl.ds(0, base)])
    pltpu.sync_copy(dst_hbm.at[pl.ds(start_blk, base)],
                    dst_v.at[pl.ds(0, base)])
    pltpu.sync_copy(w_hbm.at[pl.ds(start_blk, base)],
                    w_v.at[pl.ds(0, base)])
    if extra:
        @pl.when(wid < extra)
        def _():
            eb = start_blk + base
            pltpu.sync_copy(src_hbm.at[pl.ds(eb, 1)],
                            src_v.at[pl.ds(base, 1)])
            pltpu.sync_copy(dst_hbm.at[pl.ds(eb, 1)],
                            dst_v.at[pl.ds(base, 1)])
            pltpu.sync_copy(w_hbm.at[pl.ds(eb, 1)],
                            w_v.at[pl.ds(base, 1)])

    b0 = bufs[0]

    def _z(i, _):
        b0[i, :] = zeros16
        return 0
    lax.fori_loop(0, BLK, _z, 0)

    rows = NP // NS
    rbase = s * rows

    def _zacc(k, _):
        pltpu.sync_copy(b0, acc_sh.at[pl.ds(rbase + k * BLK, BLK)])
        return 0
    lax.fori_loop(0, rows // BLK, _zacc, 0)
    pltpu.sync_copy(h_hbm.at[pl.ds(rbase, rows)], h_sh.at[pl.ds(rbase, rows)])
    plsc.subcore_barrier()

    def start_gather(jb, buf, sem):
        pltpu.async_copy(h_sh.at[src_v.at[jb]], buf, sem)

    def wait_gather(jb, buf, sem):
        pltpu.make_async_copy(h_sh.at[src_v.at[jb]], buf, sem).wait()

    def start_scatter(jb, buf, sem):
        pltpu.async_copy(buf, acc_sh.at[dst_v.at[jb]], sem, add=True)

    def wait_scatter(jb, buf, sem):
        pltpu.make_async_copy(buf, acc_sh.at[dst_v.at[jb]], sem).wait()

    def multiply(buf, jb):
        for g in range(BLK // 16):
            wvec = w_v[jb, pl.ds(g * 16, 16)]
            for k in range(16):
                wb = jnp.take_along_axis(
                    wvec, jnp.full((16,), k, jnp.int32), axis=0)
                o = g * 16 + k
                buf[o, :] = buf[o, :] * wb

    def process(jb, r):
        s2 = (r + LEAD) % NBUF
        wait_gather(jb, bufs[r], gsems[r])
        multiply(bufs[r], jb)
        start_scatter(jb, bufs[r], ssems[r])

        @pl.when(jb + LEAD < base)
        def _():
            @pl.when(jb - LEAD >= 0)
            def _():
                wait_scatter(jb - LEAD, bufs[s2], ssems[s2])
            start_gather(jb + LEAD, bufs[s2], gsems[s2])

    for r in range(LEAD):
        start_gather(jnp.int32(r), bufs[r], gsems[r])

    def _oct(j8, _):
        for r in range(NBUF):
            process(j8 * NBUF + r, r)
        return 0
    no = base // NBUF
    lax.fori_loop(0, no, _oct, 0)
    for r in range(base % NBUF):
        process(jnp.int32(no * NBUF + r), r)
    for t in range(min(2 * LEAD, base)):
        blk = base - min(2 * LEAD, base) + t
        wait_scatter(jnp.int32(blk), bufs[blk % NBUF], ssems[blk % NBUF])

    if extra:
        @pl.when(wid < extra)
        def _():
            jb = jnp.int32(base)
            start_gather(jb, bufs[0], gsems[0])
            wait_gather(jb, bufs[0], gsems[0])
            multiply(bufs[0], jb)
            start_scatter(jb, bufs[0], ssems[0])
            wait_scatter(jb, bufs[0], ssems[0])

    plsc.subcore_barrier()
    pltpu.sync_copy(acc_sh.at[pl.ds(rbase, rows)],
                    out_hbm.at[c, pl.ds(rbase, rows)])


def _edge_aggregate(h, src, dst, w, base, extra):
    mesh = plsc.VectorSubcoreMesh(core_axis_name="c", subcore_axis_name="s")
    body = functools.partial(_sc_body, base=base, extra=extra)
    nslab = base + (1 if extra else 0)
    return pl.kernel(
        body,
        out_type=jax.ShapeDtypeStruct((NC, NP, D_PAD), jnp.float32),
        mesh=mesh,
        compiler_params=pltpu.CompilerParams(use_tc_tiling_on_sc=False),
        scratch_types=[
            [pltpu.VMEM((BLK, D_PAD), jnp.float32) for _ in range(NBUF)],
            pltpu.VMEM((nslab, BLK), jnp.int32),
            pltpu.VMEM((nslab, BLK), jnp.int32),
            pltpu.VMEM((nslab, BLK), jnp.float32),
            pltpu.VMEM_SHARED((NP, D_PAD), jnp.float32),
            pltpu.VMEM_SHARED((NP, D_PAD), jnp.float32),
            [pltpu.SemaphoreType.DMA for _ in range(NBUF)],
            [pltpu.SemaphoreType.DMA for _ in range(NBUF)],
        ],
    )(h, src, dst, w)


@jax.jit
def kernel(x, edge_index, w_mul, W, b):
    e = edge_index.shape[1]
    edges = edge_index.astype(jnp.int32)
    w = w_mul
    if e % BLK:
        pad = BLK - e % BLK
        edges = jnp.concatenate(
            [edges, jnp.zeros((2, pad), jnp.int32)], axis=1)
        w = jnp.concatenate([w, jnp.zeros((pad,), jnp.float32)])
        e += pad
    nblk = e // BLK
    base, extra = divmod(nblk, NW)
    w = w.reshape(nblk, BLK)

    w_pad = jnp.zeros((D_IN, D_PAD), jnp.float32).at[:, : W.shape[1]].set(W)
    h, src, dst = _matmul(x, w_pad, edges)

    partials = _edge_aggregate(h, src, dst, w, base, extra)
    pf = partials.reshape(NC, NP * D_PAD)
    out = (pf[0] + pf[1]).reshape(NP, D_PAD)
    return out[:N, : W.shape[1]] + b


# speedup vs baseline: 1.4542x; 1.2543x over previous
import functools

import jax
import jax.numpy as jnp
from jax import lax
from jax.experimental import pallas as pl
from jax.experimental.pallas import tpu as pltpu
from jax.experimental.pallas import tpu_sc as plsc

N = 10000
NP = 10240
D_IN = 128
D_PAD = 16

NC = 2
NS = 16
NW = NC * NS
BLK = 128


def _mm_body(x_ref, w_ref, e_ref, o_ref, src_ref, dst_ref):
    o_ref[pl.ds(0, N), :] = jnp.dot(x_ref[...], w_ref[...],
                                    preferred_element_type=jnp.float32)
    e = e_ref[...]
    nblk = e.shape[1] // BLK
    src_ref[...] = e[0].reshape(nblk, BLK)
    dst_ref[...] = e[1].reshape(nblk, BLK)


def _matmul(x, w_pad, edges):
    nblk = edges.shape[1] // BLK
    return pl.pallas_call(
        _mm_body,
        out_shape=(
            jax.ShapeDtypeStruct((NP, D_PAD), jnp.float32),
            jax.ShapeDtypeStruct((nblk, BLK), jnp.int32),
            jax.ShapeDtypeStruct((nblk, BLK), jnp.int32),
        ),
    )(x, w_pad, edges)


NBUF = 4
LEAD = 2


def _sc_body(h_hbm, src_hbm, dst_hbm, w_hbm, out_hbm,
             bufs, src_v, dst_v, w_v, h_sh, acc_sh, gsems, ssems,
             base, extra):
    c = lax.axis_index("c")
    s = lax.axis_index("s")
    wid = c * NS + s

    zeros16 = jnp.zeros((16,), jnp.float32)

    start_blk = wid * base + jnp.minimum(wid, extra)
    pltpu.sync_copy(src_hbm.at[pl.ds(start_blk, base)],
                    src_v.at[pl.ds(0, base)])
    pltpu.sync_copy(dst_hbm.at[pl.ds(start_blk, base)],
                    dst_v.at[pl.ds(0, base)])
    pltpu.sync_copy(w_hbm.at[pl.ds(start_blk, base)],
                    w_v.at[pl.ds(0, base)])
    if extra:
        @pl.when(wid < extra)
        def _():
            eb = start_blk + base
            pltpu.sync_copy(src_hbm.at[pl.ds(eb, 1)],
                            src_v.at[pl.ds(base, 1)])
            pltpu.sync_copy(dst_hbm.at[pl.ds(eb, 1)],
                            dst_v.at[pl.ds(base, 1)])
            pltpu.sync_copy(w_hbm.at[pl.ds(eb, 1)],
                            w_v.at[pl.ds(base, 1)])

    b0 = bufs[0]

    def _z(i, _):
        b0[i, :] = zeros16
        return 0
    lax.fori_loop(0, BLK, _z, 0)

    rows = NP // NS
    rbase = s * rows

    def _zacc(k, _):
        pltpu.sync_copy(b0, acc_sh.at[pl.ds(rbase + k * BLK, BLK)])
        return 0
    lax.fori_loop(0, rows // BLK, _zacc, 0)
    pltpu.sync_copy(h_hbm.at[pl.ds(rbase, rows)], h_sh.at[pl.ds(rbase, rows)])
    plsc.subcore_barrier()

    def start_gather(jb, buf, sem):
        pltpu.async_copy(h_sh.at[src_v.at[jb]], buf, sem)

    def wait_gather(jb, buf, sem):
        pltpu.make_async_copy(h_sh.at[src_v.at[jb]], buf, sem).wait()

    def start_scatter(jb, buf, sem):
        pltpu.async_copy(buf, acc_sh.at[dst_v.at[jb]], sem, add=True)

    def wait_scatter(jb, buf, sem):
        pltpu.make_async_copy(buf, acc_sh.at[dst_v.at[jb]], sem).wait()

    def multiply(buf, jb):
        for g in range(BLK // 16):
            wvec = w_v[jb, pl.ds(g * 16, 16)]
            for k in range(16):
                wb = jnp.take_along_axis(
                    wvec, jnp.full((16,), k, jnp.int32), axis=0)
                o = g * 16 + k
                buf[o, :] = buf[o, :] * wb

    def process(jb, r):
        s2 = (r + LEAD) % NBUF
        wait_gather(jb, bufs[r], gsems[r])
        multiply(bufs[r], jb)
        start_scatter(jb, bufs[r], ssems[r])

        @pl.when(jb + LEAD < base)
        def _():
            @pl.when(jb - LEAD >= 0)
            def _():
                wait_scatter(jb - LEAD, bufs[s2], ssems[s2])
            start_gather(jb + LEAD, bufs[s2], gsems[s2])

    for r in range(LEAD):
        start_gather(jnp.int32(r), bufs[r], gsems[r])

    def _oct(j8, _):
        for r in range(NBUF):
            process(j8 * NBUF + r, r)
        return 0
    no = base // NBUF
    lax.fori_loop(0, no, _oct, 0)
    for r in range(base % NBUF):
        process(jnp.int32(no * NBUF + r), r)
    for t in range(min(2 * LEAD, base)):
        blk = base - min(2 * LEAD, base) + t
        wait_scatter(jnp.int32(blk), bufs[blk % NBUF], ssems[blk % NBUF])

    if extra:
        @pl.when(wid < extra)
        def _():
            jb = jnp.int32(base)
            start_gather(jb, bufs[0], gsems[0])
            wait_gather(jb, bufs[0], gsems[0])
            multiply(bufs[0], jb)
            start_scatter(jb, bufs[0], ssems[0])
            wait_scatter(jb, bufs[0], ssems[0])

    plsc.subcore_barrier()
    pltpu.sync_copy(acc_sh.at[pl.ds(rbase, rows)],
                    out_hbm.at[c, pl.ds(rbase, rows)])


def _edge_aggregate(h, src, dst, w, base, extra):
    mesh = plsc.VectorSubcoreMesh(core_axis_name="c", subcore_axis_name="s")
    body = functools.partial(_sc_body, base=base, extra=extra)
    nslab = base + (1 if extra else 0)
    return pl.kernel(
        body,
        out_type=jax.ShapeDtypeStruct((NC, NP, D_PAD), jnp.float32),
        mesh=mesh,
        compiler_params=pltpu.CompilerParams(use_tc_tiling_on_sc=False),
        scratch_types=[
            [pltpu.VMEM((BLK, D_PAD), jnp.float32) for _ in range(NBUF)],
            pltpu.VMEM((nslab, BLK), jnp.int32),
            pltpu.VMEM((nslab, BLK), jnp.int32),
            pltpu.VMEM((nslab, BLK), jnp.float32),
            pltpu.VMEM_SHARED((NP, D_PAD), jnp.float32),
            pltpu.VMEM_SHARED((NP, D_PAD), jnp.float32),
            [pltpu.SemaphoreType.DMA for _ in range(NBUF)],
            [pltpu.SemaphoreType.DMA for _ in range(NBUF)],
        ],
    )(h, src, dst, w)


@jax.jit
def kernel(x, edge_index, w_mul, W, b):
    e = edge_index.shape[1]
    edges = edge_index.astype(jnp.int32)
    w = w_mul
    if e % BLK:
        pad = BLK - e % BLK
        edges = jnp.concatenate(
            [edges, jnp.zeros((2, pad), jnp.int32)], axis=1)
        w = jnp.concatenate([w, jnp.zeros((pad,), jnp.float32)])
        e += pad
    nblk = e // BLK
    base, extra = divmod(nblk, NW)
    w = w.reshape(nblk, BLK)

    w_pad = jnp.zeros((D_IN, D_PAD), jnp.float32).at[:, : W.shape[1]].set(W)
    h, src, dst = _matmul(x, w_pad, edges)

    partials = _edge_aggregate(h, src, dst, w, base, extra)
    d = W.shape[1]
    return partials[0, :N, :d] + partials[1, :N, :d] + b
